# baseline (speedup 1.0000x reference)
import functools

import jax
import jax.numpy as jnp
from jax import lax
from jax.experimental import pallas as pl
from jax.experimental.pallas import tpu as pltpu
from jax.experimental.pallas import tpu_sc as plsc

HID = 128
NUM_RELS = 16
N = 10000
NPAD = 10240
E = 320000
NC, NS, L = 2, 16, 16
NW = NC * NS
CW = 128
CH = 80
PP = 8
CH_A = 56
CH_B = 104
EPT = CH * CW
EPAD = NW * EPT
SLICE = NPAD // NS
NB = 512


def _node_body(x_ref, w_ref, att_ref, rel_ref, wrel_ref, c_ref,
               h_ref, s1_ref, s2_ref, rs_ref, m1_ref, m2_ref, m3_ref,
               cl_ref):
    i = pl.program_id(0)
    cl_ref[...] = 0.1 * jnp.log(jnp.maximum(c_ref[...], 1e-6))
    h = lax.dot_general(x_ref[...], w_ref[...], (((1,), (1,)), ((), ())),
                        preferred_element_type=jnp.float32)
    h_ref[...] = h
    a_dst = att_ref[0:1, :]
    a_src = att_ref[1:2, :]
    s1 = lax.dot_general(h, a_dst, (((1,), (1,)), ((), ())),
                         preferred_element_type=jnp.float32)
    s2 = lax.dot_general(h, a_src, (((1,), (1,)), ((), ())),
                         preferred_element_type=jnp.float32)
    s1_ref[...] = s1
    s2_ref[...] = s2
    neg = jnp.full((1, 1), -jnp.inf, jnp.float32)
    b1 = jnp.max(s1, keepdims=True)
    b2 = jnp.max(s2, keepdims=True)
    m1_ref[...] = jnp.maximum(jnp.where(i == 0, neg, m1_ref[...]), b1)
    m2_ref[...] = jnp.maximum(jnp.where(i == 0, neg, m2_ref[...]), b2)

    @pl.when(i == 0)
    def _():
        a_rel = att_ref[2:3, :]
        c = lax.dot_general(a_rel, wrel_ref[...], (((1,), (0,)), ((), ())),
                            preferred_element_type=jnp.float32)
        rs = lax.dot_general(rel_ref[...], c, (((1,), (1,)), ((), ())),
                             preferred_element_type=jnp.float32)
        rs_row = rs.reshape(1, 16)
        rs_ref[...] = jnp.concatenate(
            [rs_row, jnp.zeros((1, 112), jnp.float32)], axis=1)
        m3_ref[...] = jnp.max(rs, keepdims=True)


def _stage1(xp, w, att3, rel_emb, wrel, conf2d):
    grid = NPAD // NB
    return pl.pallas_call(
        _node_body,
        grid=(grid,),
        in_specs=[
            pl.BlockSpec((NB, HID), lambda i: (i, 0)),
            pl.BlockSpec((HID, HID), lambda i: (0, 0)),
            pl.BlockSpec((3, HID), lambda i: (0, 0)),
            pl.BlockSpec((NUM_RELS, NUM_RELS), lambda i: (0, 0)),
            pl.BlockSpec((HID, NUM_RELS), lambda i: (0, 0)),
            pl.BlockSpec((CW, 128), lambda i: (i, 0)),
        ],
        out_specs=[
            pl.BlockSpec((NB, HID), lambda i: (i, 0)),
            pl.BlockSpec((NB, 1), lambda i: (i, 0)),
            pl.BlockSpec((NB, 1), lambda i: (i, 0)),
            pl.BlockSpec((1, HID), lambda i: (0, 0)),
            pl.BlockSpec((1, 1), lambda i: (0, 0)),
            pl.BlockSpec((1, 1), lambda i: (0, 0)),
            pl.BlockSpec((1, 1), lambda i: (0, 0)),
            pl.BlockSpec((CW, 128), lambda i: (i, 0)),
        ],
        out_shape=[
            jax.ShapeDtypeStruct((NPAD, HID), jnp.float32),
            jax.ShapeDtypeStruct((NPAD, 1), jnp.float32),
            jax.ShapeDtypeStruct((NPAD, 1), jnp.float32),
            jax.ShapeDtypeStruct((1, HID), jnp.float32),
            jax.ShapeDtypeStruct((1, 1), jnp.float32),
            jax.ShapeDtypeStruct((1, 1), jnp.float32),
            jax.ShapeDtypeStruct((1, 1), jnp.float32),
            jax.ShapeDtypeStruct((EPAD // 128, 128), jnp.float32),
        ],
    )(xp, w, att3, rel_emb, wrel, conf2d)


def _score_body(s1_hbm, s2_hbm, rel_hbm, shift_hbm, dst_hbm, src_hbm,
                t_hbm, clog_hbm, e_hbm, sp_hbm,
                s1v, s2v, relv, shv, dstv, srcv, tv, clogv, ev, zv, s_acc):
    cid = lax.axis_index("c")
    sid = lax.axis_index("s")
    wid = cid * NS + sid

    pltpu.sync_copy(s1_hbm, s1v)
    pltpu.sync_copy(s2_hbm, s2v)
    pltpu.sync_copy(rel_hbm, relv)
    pltpu.sync_copy(shift_hbm, shv)
    pltpu.sync_copy(dst_hbm.at[wid], dstv)
    pltpu.sync_copy(src_hbm.at[wid], srcv)
    pltpu.sync_copy(t_hbm.at[wid], tv)
    pltpu.sync_copy(clog_hbm.at[wid], clogv)

    def zfill(i, _):
        zv[pl.ds(i * L, L)] = jnp.zeros((L,), jnp.float32)
        return 0
    lax.fori_loop(0, SLICE // L, zfill, 0)
    pltpu.sync_copy(zv, s_acc.at[pl.ds(sid * SLICE, SLICE)])
    plsc.subcore_barrier()

    sh = shv[...]
    lane = jnp.arange(L, dtype=jnp.int32)

    def row(r, _):
        for c in range(CW // L):
            sl = pl.ds(c * L, L)
            d16 = dstv[r, sl]
            g = (plsc.load_gather(s1v, [d16])
                 + plsc.load_gather(s2v, [srcv[r, sl]])
                 + plsc.load_gather(relv, [tv[r, sl]]))
            g = jnp.where(g >= 0.0, g, 0.2 * g)
            g = g + clogv[r, sl] - sh
            e16 = jnp.exp(g)
            gidx = wid * EPT + r * CW + c * L + lane
            ev[r, sl] = jnp.where(gidx < E, e16, 0.0)
        pltpu.sync_copy(ev.at[r], s_acc.at[dstv.at[r]], add=True)
        return 0
    lax.fori_loop(0, CH, row, 0)

    plsc.subcore_barrier()
    pltpu.sync_copy(ev, e_hbm.at[wid])
    pltpu.sync_copy(s_acc.at[pl.ds(sid * SLICE, SLICE)],
                    sp_hbm.at[cid, pl.ds(sid * SLICE, SLICE)])


def _stage3(s1, s2, rel16, shift8, dstI, srcI, tI, clogI):
    mesh = plsc.VectorSubcoreMesh(core_axis_name="c", subcore_axis_name="s",
                                  num_cores=NC, num_subcores=NS)
    f = pl.kernel(
        _score_body,
        out_type=[
            jax.ShapeDtypeStruct((NW, CH, CW), jnp.float32),
            jax.ShapeDtypeStruct((NC, NPAD), jnp.float32),
        ],
        mesh=mesh,
        compiler_params=pltpu.CompilerParams(needs_layout_passes=False),
        scratch_types=[
            pltpu.VMEM((NPAD,), jnp.float32),
            pltpu.VMEM((NPAD,), jnp.float32),
            pltpu.VMEM((NUM_RELS,), jnp.float32),
            pltpu.VMEM((L,), jnp.float32),
            pltpu.VMEM((CH, CW), jnp.int32),
            pltpu.VMEM((CH, CW), jnp.int32),
            pltpu.VMEM((CH, CW), jnp.int32),
            pltpu.VMEM((CH, CW), jnp.float32),
            pltpu.VMEM((CH, CW), jnp.float32),
            pltpu.VMEM((SLICE,), jnp.float32),
            pltpu.VMEM_SHARED((NPAD,), jnp.float32),
        ],
    )
    return f(s1, s2, rel16, shift8, dstI, srcI, tI, clogI)


def _alpha_body(sp_hbm, dst_hbm, e_hbm, a_hbm, sp0v, sp1v, dstv, ev):
    cid = lax.axis_index("c")
    sid = lax.axis_index("s")
    wid = cid * NS + sid

    pltpu.sync_copy(sp_hbm.at[0], sp0v)
    pltpu.sync_copy(sp_hbm.at[1], sp1v)
    pltpu.sync_copy(dst_hbm.at[wid], dstv)
    pltpu.sync_copy(e_hbm.at[wid], ev)

    def comb(i, _):
        sl = pl.ds(i * L, L)
        sp0v[sl] = sp0v[sl] + sp1v[sl]
        return 0
    lax.fori_loop(0, NPAD // L, comb, 0)

    def arow(r, _):
        for c in range(CW // L):
            sl = pl.ds(c * L, L)
            s16 = plsc.load_gather(sp0v, [dstv[r, sl]])
            ev[r, sl] = ev[r, sl] / (s16 + 1e-16)
        return 0
    lax.fori_loop(0, CH, arow, 0)
    pltpu.sync_copy(ev, a_hbm.at[wid])


def _stage3b(sp, dstI, e):
    mesh = plsc.VectorSubcoreMesh(core_axis_name="c", subcore_axis_name="s",
                                  num_cores=NC, num_subcores=NS)
    f = pl.kernel(
        _alpha_body,
        out_type=jax.ShapeDtypeStruct((NW, CH, CW), jnp.float32),
        mesh=mesh,
        compiler_params=pltpu.CompilerParams(needs_layout_passes=False),
        scratch_types=[
            pltpu.VMEM((NPAD,), jnp.float32),
            pltpu.VMEM((NPAD,), jnp.float32),
            pltpu.VMEM((CH, CW), jnp.int32),
            pltpu.VMEM((CH, CW), jnp.float32),
        ],
    )
    return f(sp, dstI, e)


def _msg_body(h_hbm, a_hbm, dst_hbm, src_hbm, op_hbm,
              dpv, spv, apv, rows_a, rows_b, sem_a, sem_b, out_acc):
    cid = lax.axis_index("c")
    sid = lax.axis_index("s")
    wid = cid * NS + sid

    def zrow(k, _):
        for c in range(HID // L):
            rows_a[k, pl.ds(c * L, L)] = jnp.zeros((L,), jnp.float32)
        return 0
    lax.fori_loop(0, CW, zrow, 0)

    def zacc(j, _):
        pltpu.sync_copy(rows_a, out_acc.at[pl.ds(sid * SLICE + j * CW, CW)])
        return 0
    lax.fori_loop(0, SLICE // CW, zacc, 0)
    plsc.subcore_barrier()

    def run(chunk0, nch):
        for p in range(nch // PP):
            start = pl.multiple_of(chunk0 + p * PP, PP)
            pltpu.sync_copy(dst_hbm.at[pl.ds(start, PP)], dpv)
            pltpu.sync_copy(src_hbm.at[pl.ds(start, PP)], spv)
            pltpu.sync_copy(a_hbm.at[pl.ds(start, PP)], apv)
            pltpu.async_copy(h_hbm.at[spv.at[0]], rows_a, sem_a)
            pltpu.async_copy(h_hbm.at[spv.at[1]], rows_b, sem_b)

            def pair(g, _):
                for b in range(2):
                    rows = rows_a if b == 0 else rows_b
                    sem = sem_a if b == 0 else sem_b
                    r = g * 2 + b
                    pltpu.make_async_copy(h_hbm.at[spv.at[r]], rows, sem).wait()

                    def scale(k, _2):
                        a = plsc.load_gather(
                            apv, [jnp.full((L,), r, jnp.int32),
                                  jnp.full((L,), k, jnp.int32)])
                        for c in range(HID // L):
                            sl = pl.ds(c * L, L)
                            rows[k, sl] = rows[k, sl] * a
                        return 0
                    lax.fori_loop(0, CW, scale, 0)
                    pltpu.sync_copy(rows, out_acc.at[dpv.at[r]], add=True)

                    @pl.when(r + 2 < PP)
                    def _():
                        pltpu.async_copy(h_hbm.at[spv.at[r + 2]], rows, sem)
                return 0
            lax.fori_loop(0, PP // 2, pair, 0)

    @pl.when(cid == 0)
    def _():
        run(sid * CH_A, CH_A)

    @pl.when(cid == 1)
    def _():
        run(NS * CH_A + sid * CH_B, CH_B)

    plsc.subcore_barrier()
    pltpu.sync_copy(out_acc.at[pl.ds(sid * SLICE, SLICE)],
                    op_hbm.at[cid, pl.ds(sid * SLICE, SLICE)])


def _stage4(h, alpha, dstI, srcI):
    mesh = plsc.VectorSubcoreMesh(core_axis_name="c", subcore_axis_name="s",
                                  num_cores=NC, num_subcores=NS)
    f = pl.kernel(
        _msg_body,
        out_type=jax.ShapeDtypeStruct((NC, NPAD, HID), jnp.float32),
        mesh=mesh,
        compiler_params=pltpu.CompilerParams(needs_layout_passes=False),
        scratch_types=[
            pltpu.VMEM((PP, CW), jnp.int32),
            pltpu.VMEM((PP, CW), jnp.int32),
            pltpu.VMEM((PP, CW), jnp.float32),
            pltpu.VMEM((CW, HID), jnp.float32),
            pltpu.VMEM((CW, HID), jnp.float32),
            pltpu.SemaphoreType.DMA,
            pltpu.SemaphoreType.DMA,
            pltpu.VMEM_SHARED((NPAD, HID), jnp.float32),
        ],
    )
    return f(h, alpha.reshape(EPAD // CW, CW), dstI.reshape(EPAD // CW, CW),
             srcI.reshape(EPAD // CW, CW))


def _fin_body(p0_ref, p1_ref, b_ref, o_ref):
    o_ref[...] = p0_ref[...] + p1_ref[...] + b_ref[...]


def _stage5(p0, p1, bias2d):
    return pl.pallas_call(
        _fin_body,
        grid=(NS,),
        in_specs=[
            pl.BlockSpec((SLICE, HID), lambda i: (i, 0)),
            pl.BlockSpec((SLICE, HID), lambda i: (i, 0)),
            pl.BlockSpec((1, HID), lambda i: (0, 0)),
        ],
        out_specs=pl.BlockSpec((SLICE, HID), lambda i: (i, 0)),
        out_shape=jax.ShapeDtypeStruct((NPAD, HID), jnp.float32),
    )(p0, p1, bias2d)


@jax.jit
def kernel(x, edge_index, edge_type_in, edge_attr, W_msg, rel_emb, W_rel,
           att_vec, bias):
    src = edge_index[0].astype(jnp.int32)
    dst = edge_index[1].astype(jnp.int32)
    t = jnp.clip(edge_type_in, 0, NUM_RELS - 1).astype(jnp.int32)
    conf = edge_attr[:, 0].astype(jnp.float32)

    pad = EPAD - E
    srcI = jnp.pad(src, (0, pad)).reshape(NW, CH, CW)
    dstI = jnp.pad(dst, (0, pad)).reshape(NW, CH, CW)
    tI = jnp.pad(t, (0, pad)).reshape(NW, CH, CW)
    confP = jnp.pad(conf, (0, pad), constant_values=1.0)

    xp = jnp.pad(x, ((0, NPAD - N), (0, 0)))
    att3 = att_vec.reshape(3, HID)

    h, s1, s2, rs, m1, m2, m3, clog2d = _stage1(
        xp, W_msg, att3, rel_emb, W_rel, confP.reshape(EPAD // 128, 128))
    shift = jnp.maximum(m1[0, 0] + m2[0, 0] + m3[0, 0], 0.0)
    shift16 = jnp.broadcast_to(shift, (L,))
    clog = clog2d.reshape(NW, CH, CW)

    e, sp = _stage3(s1.reshape(NPAD), s2.reshape(NPAD), rs[0, :NUM_RELS],
                    shift16, dstI, srcI, tI, clog)
    alpha = _stage3b(sp, dstI, e)
    op = _stage4(h, alpha, dstI, srcI)
    out = _stage5(op[0], op[1], bias.reshape(1, HID))
    return out[:N]


# speedup vs baseline: 1.1399x; 1.1399x over previous
import functools

import jax
import jax.numpy as jnp
from jax import lax
from jax.experimental import pallas as pl
from jax.experimental.pallas import tpu as pltpu
from jax.experimental.pallas import tpu_sc as plsc

HID = 128
NUM_RELS = 16
N = 10000
NPAD = 10240
E = 320000
NC, NS, L = 2, 16, 16
NW = NC * NS
CW = 128
CH = 80
PP = 8
CH_A = 104
CH_B = 56
EPT = CH * CW
EPAD = NW * EPT
SLICE = NPAD // NS
NB = 512


def _node_body(x_ref, w_ref, att_ref, rel_ref, wrel_ref, c_ref,
               h_ref, s1_ref, s2_ref, rs_ref, m1_ref, m2_ref, m3_ref,
               cl_ref):
    i = pl.program_id(0)
    cl_ref[...] = 0.1 * jnp.log(jnp.maximum(c_ref[...], 1e-6))
    h = lax.dot_general(x_ref[...], w_ref[...], (((1,), (1,)), ((), ())),
                        preferred_element_type=jnp.float32)
    h_ref[...] = h
    a_dst = att_ref[0:1, :]
    a_src = att_ref[1:2, :]
    s1 = lax.dot_general(h, a_dst, (((1,), (1,)), ((), ())),
                         preferred_element_type=jnp.float32)
    s2 = lax.dot_general(h, a_src, (((1,), (1,)), ((), ())),
                         preferred_element_type=jnp.float32)
    s1_ref[...] = s1
    s2_ref[...] = s2
    neg = jnp.full((1, 1), -jnp.inf, jnp.float32)
    b1 = jnp.max(s1, keepdims=True)
    b2 = jnp.max(s2, keepdims=True)
    m1_ref[...] = jnp.maximum(jnp.where(i == 0, neg, m1_ref[...]), b1)
    m2_ref[...] = jnp.maximum(jnp.where(i == 0, neg, m2_ref[...]), b2)

    @pl.when(i == 0)
    def _():
        a_rel = att_ref[2:3, :]
        c = lax.dot_general(a_rel, wrel_ref[...], (((1,), (0,)), ((), ())),
                            preferred_element_type=jnp.float32)
        rs = lax.dot_general(rel_ref[...], c, (((1,), (1,)), ((), ())),
                             preferred_element_type=jnp.float32)
        rs_row = rs.reshape(1, 16)
        rs_ref[...] = jnp.concatenate(
            [rs_row, jnp.zeros((1, 112), jnp.float32)], axis=1)
        m3_ref[...] = jnp.max(rs, keepdims=True)


def _stage1(xp, w, att3, rel_emb, wrel, conf2d):
    grid = NPAD // NB
    return pl.pallas_call(
        _node_body,
        grid=(grid,),
        in_specs=[
            pl.BlockSpec((NB, HID), lambda i: (i, 0)),
            pl.BlockSpec((HID, HID), lambda i: (0, 0)),
            pl.BlockSpec((3, HID), lambda i: (0, 0)),
            pl.BlockSpec((NUM_RELS, NUM_RELS), lambda i: (0, 0)),
            pl.BlockSpec((HID, NUM_RELS), lambda i: (0, 0)),
            pl.BlockSpec((CW, 128), lambda i: (i, 0)),
        ],
        out_specs=[
            pl.BlockSpec((NB, HID), lambda i: (i, 0)),
            pl.BlockSpec((NB, 1), lambda i: (i, 0)),
            pl.BlockSpec((NB, 1), lambda i: (i, 0)),
            pl.BlockSpec((1, HID), lambda i: (0, 0)),
            pl.BlockSpec((1, 1), lambda i: (0, 0)),
            pl.BlockSpec((1, 1), lambda i: (0, 0)),
            pl.BlockSpec((1, 1), lambda i: (0, 0)),
            pl.BlockSpec((CW, 128), lambda i: (i, 0)),
        ],
        out_shape=[
            jax.ShapeDtypeStruct((NPAD, HID), jnp.float32),
            jax.ShapeDtypeStruct((NPAD, 1), jnp.float32),
            jax.ShapeDtypeStruct((NPAD, 1), jnp.float32),
            jax.ShapeDtypeStruct((1, HID), jnp.float32),
            jax.ShapeDtypeStruct((1, 1), jnp.float32),
            jax.ShapeDtypeStruct((1, 1), jnp.float32),
            jax.ShapeDtypeStruct((1, 1), jnp.float32),
            jax.ShapeDtypeStruct((EPAD // 128, 128), jnp.float32),
        ],
    )(xp, w, att3, rel_emb, wrel, conf2d)


def _score_body(s1_hbm, s2_hbm, rel_hbm, shift_hbm, dst_hbm, src_hbm,
                t_hbm, clog_hbm, e_hbm, sp_hbm,
                s1v, s2v, relv, shv, dstv, srcv, tv, clogv, ev, zv, s_acc):
    cid = lax.axis_index("c")
    sid = lax.axis_index("s")
    wid = cid * NS + sid

    pltpu.sync_copy(s1_hbm, s1v)
    pltpu.sync_copy(s2_hbm, s2v)
    pltpu.sync_copy(rel_hbm, relv)
    pltpu.sync_copy(shift_hbm, shv)
    pltpu.sync_copy(dst_hbm.at[wid], dstv)
    pltpu.sync_copy(src_hbm.at[wid], srcv)
    pltpu.sync_copy(t_hbm.at[wid], tv)
    pltpu.sync_copy(clog_hbm.at[wid], clogv)

    def zfill(i, _):
        zv[pl.ds(i * L, L)] = jnp.zeros((L,), jnp.float32)
        return 0
    lax.fori_loop(0, SLICE // L, zfill, 0)
    pltpu.sync_copy(zv, s_acc.at[pl.ds(sid * SLICE, SLICE)])
    plsc.subcore_barrier()

    sh = shv[...]
    lane = jnp.arange(L, dtype=jnp.int32)

    def row(r, _):
        for c in range(CW // L):
            sl = pl.ds(c * L, L)
            d16 = dstv[r, sl]
            g = (plsc.load_gather(s1v, [d16])
                 + plsc.load_gather(s2v, [srcv[r, sl]])
                 + plsc.load_gather(relv, [tv[r, sl]]))
            g = jnp.where(g >= 0.0, g, 0.2 * g)
            g = g + clogv[r, sl] - sh
            e16 = jnp.exp(g)
            gidx = wid * EPT + r * CW + c * L + lane
            ev[r, sl] = jnp.where(gidx < E, e16, 0.0)
        pltpu.sync_copy(ev.at[r], s_acc.at[dstv.at[r]], add=True)
        return 0
    lax.fori_loop(0, CH, row, 0)

    plsc.subcore_barrier()
    pltpu.sync_copy(ev, e_hbm.at[wid])
    pltpu.sync_copy(s_acc.at[pl.ds(sid * SLICE, SLICE)],
                    sp_hbm.at[cid, pl.ds(sid * SLICE, SLICE)])


def _stage3(s1, s2, rel16, shift8, dstI, srcI, tI, clogI):
    mesh = plsc.VectorSubcoreMesh(core_axis_name="c", subcore_axis_name="s",
                                  num_cores=NC, num_subcores=NS)
    f = pl.kernel(
        _score_body,
        out_type=[
            jax.ShapeDtypeStruct((NW, CH, CW), jnp.float32),
            jax.ShapeDtypeStruct((NC, NPAD), jnp.float32),
        ],
        mesh=mesh,
        compiler_params=pltpu.CompilerParams(needs_layout_passes=False),
        scratch_types=[
            pltpu.VMEM((NPAD,), jnp.float32),
            pltpu.VMEM((NPAD,), jnp.float32),
            pltpu.VMEM((NUM_RELS,), jnp.float32),
            pltpu.VMEM((L,), jnp.float32),
            pltpu.VMEM((CH, CW), jnp.int32),
            pltpu.VMEM((CH, CW), jnp.int32),
            pltpu.VMEM((CH, CW), jnp.int32),
            pltpu.VMEM((CH, CW), jnp.float32),
            pltpu.VMEM((CH, CW), jnp.float32),
            pltpu.VMEM((SLICE,), jnp.float32),
            pltpu.VMEM_SHARED((NPAD,), jnp.float32),
        ],
    )
    return f(s1, s2, rel16, shift8, dstI, srcI, tI, clogI)


def _alpha_body(sp_hbm, dst_hbm, e_hbm, a_hbm, sp0v, sp1v, dstv, ev):
    cid = lax.axis_index("c")
    sid = lax.axis_index("s")
    wid = cid * NS + sid

    pltpu.sync_copy(sp_hbm.at[0], sp0v)
    pltpu.sync_copy(sp_hbm.at[1], sp1v)
    pltpu.sync_copy(dst_hbm.at[wid], dstv)
    pltpu.sync_copy(e_hbm.at[wid], ev)

    def comb(i, _):
        sl = pl.ds(i * L, L)
        sp0v[sl] = sp0v[sl] + sp1v[sl]
        return 0
    lax.fori_loop(0, NPAD // L, comb, 0)

    def arow(r, _):
        for c in range(CW // L):
            sl = pl.ds(c * L, L)
            s16 = plsc.load_gather(sp0v, [dstv[r, sl]])
            ev[r, sl] = ev[r, sl] / (s16 + 1e-16)
        return 0
    lax.fori_loop(0, CH, arow, 0)
    pltpu.sync_copy(ev, a_hbm.at[wid])


def _stage3b(sp, dstI, e):
    mesh = plsc.VectorSubcoreMesh(core_axis_name="c", subcore_axis_name="s",
                                  num_cores=NC, num_subcores=NS)
    f = pl.kernel(
        _alpha_body,
        out_type=jax.ShapeDtypeStruct((NW, CH, CW), jnp.float32),
        mesh=mesh,
        compiler_params=pltpu.CompilerParams(needs_layout_passes=False),
        scratch_types=[
            pltpu.VMEM((NPAD,), jnp.float32),
            pltpu.VMEM((NPAD,), jnp.float32),
            pltpu.VMEM((CH, CW), jnp.int32),
            pltpu.VMEM((CH, CW), jnp.float32),
        ],
    )
    return f(sp, dstI, e)


def _msg_body(h_hbm, a_hbm, dst_hbm, src_hbm, op_hbm,
              dpv, spv, apv, rows_a, rows_b, sem_a, sem_b, out_acc):
    cid = lax.axis_index("c")
    sid = lax.axis_index("s")
    wid = cid * NS + sid

    def zrow(k, _):
        for c in range(HID // L):
            rows_a[k, pl.ds(c * L, L)] = jnp.zeros((L,), jnp.float32)
        return 0
    lax.fori_loop(0, CW, zrow, 0)

    def zacc(j, _):
        pltpu.sync_copy(rows_a, out_acc.at[pl.ds(sid * SLICE + j * CW, CW)])
        return 0
    lax.fori_loop(0, SLICE // CW, zacc, 0)
    plsc.subcore_barrier()

    def run(chunk0, nch):
        for p in range(nch // PP):
            start = pl.multiple_of(chunk0 + p * PP, PP)
            pltpu.sync_copy(dst_hbm.at[pl.ds(start, PP)], dpv)
            pltpu.sync_copy(src_hbm.at[pl.ds(start, PP)], spv)
            pltpu.sync_copy(a_hbm.at[pl.ds(start, PP)], apv)
            pltpu.async_copy(h_hbm.at[spv.at[0]], rows_a, sem_a)
            pltpu.async_copy(h_hbm.at[spv.at[1]], rows_b, sem_b)

            def pair(g, _):
                for b in range(2):
                    rows = rows_a if b == 0 else rows_b
                    sem = sem_a if b == 0 else sem_b
                    r = g * 2 + b
                    pltpu.make_async_copy(h_hbm.at[spv.at[r]], rows, sem).wait()

                    def scale(k, _2):
                        a = plsc.load_gather(
                            apv, [jnp.full((L,), r, jnp.int32),
                                  jnp.full((L,), k, jnp.int32)])
                        for c in range(HID // L):
                            sl = pl.ds(c * L, L)
                            rows[k, sl] = rows[k, sl] * a
                        return 0
                    lax.fori_loop(0, CW, scale, 0)
                    pltpu.sync_copy(rows, out_acc.at[dpv.at[r]], add=True)

                    @pl.when(r + 2 < PP)
                    def _():
                        pltpu.async_copy(h_hbm.at[spv.at[r + 2]], rows, sem)
                return 0
            lax.fori_loop(0, PP // 2, pair, 0)

    @pl.when(cid == 0)
    def _():
        run(sid * CH_A, CH_A)

    @pl.when(cid == 1)
    def _():
        run(NS * CH_A + sid * CH_B, CH_B)

    plsc.subcore_barrier()
    pltpu.sync_copy(out_acc.at[pl.ds(sid * SLICE, SLICE)],
                    op_hbm.at[cid, pl.ds(sid * SLICE, SLICE)])


def _stage4(h, alpha, dstI, srcI):
    mesh = plsc.VectorSubcoreMesh(core_axis_name="c", subcore_axis_name="s",
                                  num_cores=NC, num_subcores=NS)
    f = pl.kernel(
        _msg_body,
        out_type=jax.ShapeDtypeStruct((NC, NPAD, HID), jnp.float32),
        mesh=mesh,
        compiler_params=pltpu.CompilerParams(needs_layout_passes=False),
        scratch_types=[
            pltpu.VMEM((PP, CW), jnp.int32),
            pltpu.VMEM((PP, CW), jnp.int32),
            pltpu.VMEM((PP, CW), jnp.float32),
            pltpu.VMEM((CW, HID), jnp.float32),
            pltpu.VMEM((CW, HID), jnp.float32),
            pltpu.SemaphoreType.DMA,
            pltpu.SemaphoreType.DMA,
            pltpu.VMEM_SHARED((NPAD, HID), jnp.float32),
        ],
    )
    return f(h, alpha.reshape(EPAD // CW, CW), dstI.reshape(EPAD // CW, CW),
             srcI.reshape(EPAD // CW, CW))


def _fin_body(p0_ref, p1_ref, b_ref, o_ref):
    o_ref[...] = p0_ref[...] + p1_ref[...] + b_ref[...]


def _stage5(p0, p1, bias2d):
    return pl.pallas_call(
        _fin_body,
        grid=(NS,),
        in_specs=[
            pl.BlockSpec((SLICE, HID), lambda i: (i, 0)),
            pl.BlockSpec((SLICE, HID), lambda i: (i, 0)),
            pl.BlockSpec((1, HID), lambda i: (0, 0)),
        ],
        out_specs=pl.BlockSpec((SLICE, HID), lambda i: (i, 0)),
        out_shape=jax.ShapeDtypeStruct((NPAD, HID), jnp.float32),
    )(p0, p1, bias2d)


@jax.jit
def kernel(x, edge_index, edge_type_in, edge_attr, W_msg, rel_emb, W_rel,
           att_vec, bias):
    src = edge_index[0].astype(jnp.int32)
    dst = edge_index[1].astype(jnp.int32)
    t = jnp.clip(edge_type_in, 0, NUM_RELS - 1).astype(jnp.int32)
    conf = edge_attr[:, 0].astype(jnp.float32)

    pad = EPAD - E
    srcI = jnp.pad(src, (0, pad)).reshape(NW, CH, CW)
    dstI = jnp.pad(dst, (0, pad)).reshape(NW, CH, CW)
    tI = jnp.pad(t, (0, pad)).reshape(NW, CH, CW)
    confP = jnp.pad(conf, (0, pad), constant_values=1.0)

    xp = jnp.pad(x, ((0, NPAD - N), (0, 0)))
    att3 = att_vec.reshape(3, HID)

    h, s1, s2, rs, m1, m2, m3, clog2d = _stage1(
        xp, W_msg, att3, rel_emb, W_rel, confP.reshape(EPAD // 128, 128))
    shift = jnp.maximum(m1[0, 0] + m2[0, 0] + m3[0, 0], 0.0)
    shift16 = jnp.broadcast_to(shift, (L,))
    clog = clog2d.reshape(NW, CH, CW)

    e, sp = _stage3(s1.reshape(NPAD), s2.reshape(NPAD), rs[0, :NUM_RELS],
                    shift16, dstI, srcI, tI, clog)
    alpha = _stage3b(sp, dstI, e)
    op = _stage4(h, alpha, dstI, srcI)
    out = _stage5(op[0], op[1], bias.reshape(1, HID))
    return out[:N]


# speedup vs baseline: 1.1627x; 1.0200x over previous
import functools

import jax
import jax.numpy as jnp
from jax import lax
from jax.experimental import pallas as pl
from jax.experimental.pallas import tpu as pltpu
from jax.experimental.pallas import tpu_sc as plsc

HID = 128
NUM_RELS = 16
N = 10000
NPAD = 10240
E = 320000
NC, NS, L = 2, 16, 16
NW = NC * NS
CW = 128
CH = 80
PP = 8
CH_A = 112
CH_B = 48
EPT = CH * CW
EPAD = NW * EPT
SLICE = NPAD // NS
NB = 512


def _node_body(x_ref, w_ref, att_ref, rel_ref, wrel_ref, c_ref,
               h_ref, s1_ref, s2_ref, rs_ref, m1_ref, m2_ref, m3_ref,
               cl_ref):
    i = pl.program_id(0)
    cl_ref[...] = 0.1 * jnp.log(jnp.maximum(c_ref[...], 1e-6))
    h = lax.dot_general(x_ref[...], w_ref[...], (((1,), (1,)), ((), ())),
                        preferred_element_type=jnp.float32)
    h_ref[...] = h
    a_dst = att_ref[0:1, :]
    a_src = att_ref[1:2, :]
    s1 = lax.dot_general(h, a_dst, (((1,), (1,)), ((), ())),
                         preferred_element_type=jnp.float32)
    s2 = lax.dot_general(h, a_src, (((1,), (1,)), ((), ())),
                         preferred_element_type=jnp.float32)
    s1_ref[...] = s1
    s2_ref[...] = s2
    neg = jnp.full((1, 1), -jnp.inf, jnp.float32)
    b1 = jnp.max(s1, keepdims=True)
    b2 = jnp.max(s2, keepdims=True)
    m1_ref[...] = jnp.maximum(jnp.where(i == 0, neg, m1_ref[...]), b1)
    m2_ref[...] = jnp.maximum(jnp.where(i == 0, neg, m2_ref[...]), b2)

    @pl.when(i == 0)
    def _():
        a_rel = att_ref[2:3, :]
        c = lax.dot_general(a_rel, wrel_ref[...], (((1,), (0,)), ((), ())),
                            preferred_element_type=jnp.float32)
        rs = lax.dot_general(rel_ref[...], c, (((1,), (1,)), ((), ())),
                             preferred_element_type=jnp.float32)
        rs_row = rs.reshape(1, 16)
        rs_ref[...] = jnp.concatenate(
            [rs_row, jnp.zeros((1, 112), jnp.float32)], axis=1)
        m3_ref[...] = jnp.max(rs, keepdims=True)


def _stage1(xp, w, att3, rel_emb, wrel, conf2d):
    grid = NPAD // NB
    return pl.pallas_call(
        _node_body,
        grid=(grid,),
        in_specs=[
            pl.BlockSpec((NB, HID), lambda i: (i, 0)),
            pl.BlockSpec((HID, HID), lambda i: (0, 0)),
            pl.BlockSpec((3, HID), lambda i: (0, 0)),
            pl.BlockSpec((NUM_RELS, NUM_RELS), lambda i: (0, 0)),
            pl.BlockSpec((HID, NUM_RELS), lambda i: (0, 0)),
            pl.BlockSpec((CW, 128), lambda i: (i, 0)),
        ],
        out_specs=[
            pl.BlockSpec((NB, HID), lambda i: (i, 0)),
            pl.BlockSpec((NB, 1), lambda i: (i, 0)),
            pl.BlockSpec((NB, 1), lambda i: (i, 0)),
            pl.BlockSpec((1, HID), lambda i: (0, 0)),
            pl.BlockSpec((1, 1), lambda i: (0, 0)),
            pl.BlockSpec((1, 1), lambda i: (0, 0)),
            pl.BlockSpec((1, 1), lambda i: (0, 0)),
            pl.BlockSpec((CW, 128), lambda i: (i, 0)),
        ],
        out_shape=[
            jax.ShapeDtypeStruct((NPAD, HID), jnp.float32),
            jax.ShapeDtypeStruct((NPAD, 1), jnp.float32),
            jax.ShapeDtypeStruct((NPAD, 1), jnp.float32),
            jax.ShapeDtypeStruct((1, HID), jnp.float32),
            jax.ShapeDtypeStruct((1, 1), jnp.float32),
            jax.ShapeDtypeStruct((1, 1), jnp.float32),
            jax.ShapeDtypeStruct((1, 1), jnp.float32),
            jax.ShapeDtypeStruct((EPAD // 128, 128), jnp.float32),
        ],
    )(xp, w, att3, rel_emb, wrel, conf2d)


def _score_body(s1_hbm, s2_hbm, rel_hbm, shift_hbm, dst_hbm, src_hbm,
                t_hbm, clog_hbm, e_hbm, sp_hbm,
                s1v, s2v, relv, shv, dstv, srcv, tv, clogv, ev, zv, s_acc):
    cid = lax.axis_index("c")
    sid = lax.axis_index("s")
    wid = cid * NS + sid

    pltpu.sync_copy(s1_hbm, s1v)
    pltpu.sync_copy(s2_hbm, s2v)
    pltpu.sync_copy(rel_hbm, relv)
    pltpu.sync_copy(shift_hbm, shv)
    pltpu.sync_copy(dst_hbm.at[wid], dstv)
    pltpu.sync_copy(src_hbm.at[wid], srcv)
    pltpu.sync_copy(t_hbm.at[wid], tv)
    pltpu.sync_copy(clog_hbm.at[wid], clogv)

    def zfill(i, _):
        zv[pl.ds(i * L, L)] = jnp.zeros((L,), jnp.float32)
        return 0
    lax.fori_loop(0, SLICE // L, zfill, 0)
    pltpu.sync_copy(zv, s_acc.at[pl.ds(sid * SLICE, SLICE)])
    plsc.subcore_barrier()

    sh = shv[...]
    lane = jnp.arange(L, dtype=jnp.int32)

    def row(r, _):
        for c in range(CW // L):
            sl = pl.ds(c * L, L)
            d16 = dstv[r, sl]
            g = (plsc.load_gather(s1v, [d16])
                 + plsc.load_gather(s2v, [srcv[r, sl]])
                 + plsc.load_gather(relv, [tv[r, sl]]))
            g = jnp.where(g >= 0.0, g, 0.2 * g)
            g = g + clogv[r, sl] - sh
            e16 = jnp.exp(g)
            gidx = wid * EPT + r * CW + c * L + lane
            ev[r, sl] = jnp.where(gidx < E, e16, 0.0)
        pltpu.sync_copy(ev.at[r], s_acc.at[dstv.at[r]], add=True)
        return 0
    lax.fori_loop(0, CH, row, 0)

    plsc.subcore_barrier()
    pltpu.sync_copy(ev, e_hbm.at[wid])
    pltpu.sync_copy(s_acc.at[pl.ds(sid * SLICE, SLICE)],
                    sp_hbm.at[cid, pl.ds(sid * SLICE, SLICE)])


def _stage3(s1, s2, rel16, shift8, dstI, srcI, tI, clogI):
    mesh = plsc.VectorSubcoreMesh(core_axis_name="c", subcore_axis_name="s",
                                  num_cores=NC, num_subcores=NS)
    f = pl.kernel(
        _score_body,
        out_type=[
            jax.ShapeDtypeStruct((NW, CH, CW), jnp.float32),
            jax.ShapeDtypeStruct((NC, NPAD), jnp.float32),
        ],
        mesh=mesh,
        compiler_params=pltpu.CompilerParams(needs_layout_passes=False),
        scratch_types=[
            pltpu.VMEM((NPAD,), jnp.float32),
            pltpu.VMEM((NPAD,), jnp.float32),
            pltpu.VMEM((NUM_RELS,), jnp.float32),
            pltpu.VMEM((L,), jnp.float32),
            pltpu.VMEM((CH, CW), jnp.int32),
            pltpu.VMEM((CH, CW), jnp.int32),
            pltpu.VMEM((CH, CW), jnp.int32),
            pltpu.VMEM((CH, CW), jnp.float32),
            pltpu.VMEM((CH, CW), jnp.float32),
            pltpu.VMEM((SLICE,), jnp.float32),
            pltpu.VMEM_SHARED((NPAD,), jnp.float32),
        ],
    )
    return f(s1, s2, rel16, shift8, dstI, srcI, tI, clogI)


def _alpha_body(sp_hbm, dst_hbm, e_hbm, a_hbm, sp0v, sp1v, dstv, ev):
    cid = lax.axis_index("c")
    sid = lax.axis_index("s")
    wid = cid * NS + sid

    pltpu.sync_copy(sp_hbm.at[0], sp0v)
    pltpu.sync_copy(sp_hbm.at[1], sp1v)
    pltpu.sync_copy(dst_hbm.at[wid], dstv)
    pltpu.sync_copy(e_hbm.at[wid], ev)

    def comb(i, _):
        sl = pl.ds(i * L, L)
        sp0v[sl] = sp0v[sl] + sp1v[sl]
        return 0
    lax.fori_loop(0, NPAD // L, comb, 0)

    def arow(r, _):
        for c in range(CW // L):
            sl = pl.ds(c * L, L)
            s16 = plsc.load_gather(sp0v, [dstv[r, sl]])
            ev[r, sl] = ev[r, sl] / (s16 + 1e-16)
        return 0
    lax.fori_loop(0, CH, arow, 0)
    pltpu.sync_copy(ev, a_hbm.at[wid])


def _stage3b(sp, dstI, e):
    mesh = plsc.VectorSubcoreMesh(core_axis_name="c", subcore_axis_name="s",
                                  num_cores=NC, num_subcores=NS)
    f = pl.kernel(
        _alpha_body,
        out_type=jax.ShapeDtypeStruct((NW, CH, CW), jnp.float32),
        mesh=mesh,
        compiler_params=pltpu.CompilerParams(needs_layout_passes=False),
        scratch_types=[
            pltpu.VMEM((NPAD,), jnp.float32),
            pltpu.VMEM((NPAD,), jnp.float32),
            pltpu.VMEM((CH, CW), jnp.int32),
            pltpu.VMEM((CH, CW), jnp.float32),
        ],
    )
    return f(sp, dstI, e)


def _msg_body(h_hbm, a_hbm, dst_hbm, src_hbm, op_hbm,
              dpv, spv, apv, rows_a, rows_b, sem_a, sem_b, out_acc):
    cid = lax.axis_index("c")
    sid = lax.axis_index("s")
    wid = cid * NS + sid

    def zrow(k, _):
        for c in range(HID // L):
            rows_a[k, pl.ds(c * L, L)] = jnp.zeros((L,), jnp.float32)
        return 0
    lax.fori_loop(0, CW, zrow, 0)

    def zacc(j, _):
        pltpu.sync_copy(rows_a, out_acc.at[pl.ds(sid * SLICE + j * CW, CW)])
        return 0
    lax.fori_loop(0, SLICE // CW, zacc, 0)
    plsc.subcore_barrier()

    def run(chunk0, nch):
        for p in range(nch // PP):
            start = pl.multiple_of(chunk0 + p * PP, PP)
            pltpu.sync_copy(dst_hbm.at[pl.ds(start, PP)], dpv)
            pltpu.sync_copy(src_hbm.at[pl.ds(start, PP)], spv)
            pltpu.sync_copy(a_hbm.at[pl.ds(start, PP)], apv)
            pltpu.async_copy(h_hbm.at[spv.at[0]], rows_a, sem_a)
            pltpu.async_copy(h_hbm.at[spv.at[1]], rows_b, sem_b)

            def pair(g, _):
                for b in range(2):
                    rows = rows_a if b == 0 else rows_b
                    sem = sem_a if b == 0 else sem_b
                    r = g * 2 + b
                    pltpu.make_async_copy(h_hbm.at[spv.at[r]], rows, sem).wait()

                    def scale(k, _2):
                        a = plsc.load_gather(
                            apv, [jnp.full((L,), r, jnp.int32),
                                  jnp.full((L,), k, jnp.int32)])
                        for c in range(HID // L):
                            sl = pl.ds(c * L, L)
                            rows[k, sl] = rows[k, sl] * a
                        return 0
                    lax.fori_loop(0, CW, scale, 0)
                    pltpu.sync_copy(rows, out_acc.at[dpv.at[r]], add=True)

                    @pl.when(r + 2 < PP)
                    def _():
                        pltpu.async_copy(h_hbm.at[spv.at[r + 2]], rows, sem)
                return 0
            lax.fori_loop(0, PP // 2, pair, 0)

    @pl.when(cid == 0)
    def _():
        run(sid * CH_A, CH_A)

    @pl.when(cid == 1)
    def _():
        run(NS * CH_A + sid * CH_B, CH_B)

    plsc.subcore_barrier()
    pltpu.sync_copy(out_acc.at[pl.ds(sid * SLICE, SLICE)],
                    op_hbm.at[cid, pl.ds(sid * SLICE, SLICE)])


def _stage4(h, alpha, dstI, srcI):
    mesh = plsc.VectorSubcoreMesh(core_axis_name="c", subcore_axis_name="s",
                                  num_cores=NC, num_subcores=NS)
    f = pl.kernel(
        _msg_body,
        out_type=jax.ShapeDtypeStruct((NC, NPAD, HID), jnp.float32),
        mesh=mesh,
        compiler_params=pltpu.CompilerParams(needs_layout_passes=False),
        scratch_types=[
            pltpu.VMEM((PP, CW), jnp.int32),
            pltpu.VMEM((PP, CW), jnp.int32),
            pltpu.VMEM((PP, CW), jnp.float32),
            pltpu.VMEM((CW, HID), jnp.float32),
            pltpu.VMEM((CW, HID), jnp.float32),
            pltpu.SemaphoreType.DMA,
            pltpu.SemaphoreType.DMA,
            pltpu.VMEM_SHARED((NPAD, HID), jnp.float32),
        ],
    )
    return f(h, alpha.reshape(EPAD // CW, CW), dstI.reshape(EPAD // CW, CW),
             srcI.reshape(EPAD // CW, CW))


def _fin_body(p0_ref, p1_ref, b_ref, o_ref):
    o_ref[...] = p0_ref[...] + p1_ref[...] + b_ref[...]


def _stage5(p0, p1, bias2d):
    return pl.pallas_call(
        _fin_body,
        grid=(NS,),
        in_specs=[
            pl.BlockSpec((SLICE, HID), lambda i: (i, 0)),
            pl.BlockSpec((SLICE, HID), lambda i: (i, 0)),
            pl.BlockSpec((1, HID), lambda i: (0, 0)),
        ],
        out_specs=pl.BlockSpec((SLICE, HID), lambda i: (i, 0)),
        out_shape=jax.ShapeDtypeStruct((NPAD, HID), jnp.float32),
    )(p0, p1, bias2d)


@jax.jit
def kernel(x, edge_index, edge_type_in, edge_attr, W_msg, rel_emb, W_rel,
           att_vec, bias):
    src = edge_index[0].astype(jnp.int32)
    dst = edge_index[1].astype(jnp.int32)
    t = jnp.clip(edge_type_in, 0, NUM_RELS - 1).astype(jnp.int32)
    conf = edge_attr[:, 0].astype(jnp.float32)

    pad = EPAD - E
    srcI = jnp.pad(src, (0, pad)).reshape(NW, CH, CW)
    dstI = jnp.pad(dst, (0, pad)).reshape(NW, CH, CW)
    tI = jnp.pad(t, (0, pad)).reshape(NW, CH, CW)
    confP = jnp.pad(conf, (0, pad), constant_values=1.0)

    xp = jnp.pad(x, ((0, NPAD - N), (0, 0)))
    att3 = att_vec.reshape(3, HID)

    h, s1, s2, rs, m1, m2, m3, clog2d = _stage1(
        xp, W_msg, att3, rel_emb, W_rel, confP.reshape(EPAD // 128, 128))
    shift = jnp.maximum(m1[0, 0] + m2[0, 0] + m3[0, 0], 0.0)
    shift16 = jnp.broadcast_to(shift, (L,))
    clog = clog2d.reshape(NW, CH, CW)

    e, sp = _stage3(s1.reshape(NPAD), s2.reshape(NPAD), rs[0, :NUM_RELS],
                    shift16, dstI, srcI, tI, clog)
    alpha = _stage3b(sp, dstI, e)
    op = _stage4(h, alpha, dstI, srcI)
    out = _stage5(op[0], op[1], bias.reshape(1, HID))
    return out[:N]


# speedup vs baseline: 1.1747x; 1.0103x over previous
import functools

import jax
import jax.numpy as jnp
from jax import lax
from jax.experimental import pallas as pl
from jax.experimental.pallas import tpu as pltpu
from jax.experimental.pallas import tpu_sc as plsc

HID = 128
NUM_RELS = 16
N = 10000
NPAD = 10240
E = 320000
NC, NS, L = 2, 16, 16
NW = NC * NS
CW = 128
CH = 80
PP = 8
CH_A = 120
CH_B = 40
EPT = CH * CW
EPAD = NW * EPT
SLICE = NPAD // NS
NB = 512


def _node_body(x_ref, w_ref, att_ref, rel_ref, wrel_ref, c_ref,
               h_ref, s1_ref, s2_ref, rs_ref, m1_ref, m2_ref, m3_ref,
               cl_ref):
    i = pl.program_id(0)
    cl_ref[...] = 0.1 * jnp.log(jnp.maximum(c_ref[...], 1e-6))
    h = lax.dot_general(x_ref[...], w_ref[...], (((1,), (1,)), ((), ())),
                        preferred_element_type=jnp.float32)
    h_ref[...] = h
    a_dst = att_ref[0:1, :]
    a_src = att_ref[1:2, :]
    s1 = lax.dot_general(h, a_dst, (((1,), (1,)), ((), ())),
                         preferred_element_type=jnp.float32)
    s2 = lax.dot_general(h, a_src, (((1,), (1,)), ((), ())),
                         preferred_element_type=jnp.float32)
    s1_ref[...] = s1
    s2_ref[...] = s2
    neg = jnp.full((1, 1), -jnp.inf, jnp.float32)
    b1 = jnp.max(s1, keepdims=True)
    b2 = jnp.max(s2, keepdims=True)
    m1_ref[...] = jnp.maximum(jnp.where(i == 0, neg, m1_ref[...]), b1)
    m2_ref[...] = jnp.maximum(jnp.where(i == 0, neg, m2_ref[...]), b2)

    @pl.when(i == 0)
    def _():
        a_rel = att_ref[2:3, :]
        c = lax.dot_general(a_rel, wrel_ref[...], (((1,), (0,)), ((), ())),
                            preferred_element_type=jnp.float32)
        rs = lax.dot_general(rel_ref[...], c, (((1,), (1,)), ((), ())),
                             preferred_element_type=jnp.float32)
        rs_row = rs.reshape(1, 16)
        rs_ref[...] = jnp.concatenate(
            [rs_row, jnp.zeros((1, 112), jnp.float32)], axis=1)
        m3_ref[...] = jnp.max(rs, keepdims=True)


def _stage1(xp, w, att3, rel_emb, wrel, conf2d):
    grid = NPAD // NB
    return pl.pallas_call(
        _node_body,
        grid=(grid,),
        in_specs=[
            pl.BlockSpec((NB, HID), lambda i: (i, 0)),
            pl.BlockSpec((HID, HID), lambda i: (0, 0)),
            pl.BlockSpec((3, HID), lambda i: (0, 0)),
            pl.BlockSpec((NUM_RELS, NUM_RELS), lambda i: (0, 0)),
            pl.BlockSpec((HID, NUM_RELS), lambda i: (0, 0)),
            pl.BlockSpec((CW, 128), lambda i: (i, 0)),
        ],
        out_specs=[
            pl.BlockSpec((NB, HID), lambda i: (i, 0)),
            pl.BlockSpec((NB, 1), lambda i: (i, 0)),
            pl.BlockSpec((NB, 1), lambda i: (i, 0)),
            pl.BlockSpec((1, HID), lambda i: (0, 0)),
            pl.BlockSpec((1, 1), lambda i: (0, 0)),
            pl.BlockSpec((1, 1), lambda i: (0, 0)),
            pl.BlockSpec((1, 1), lambda i: (0, 0)),
            pl.BlockSpec((CW, 128), lambda i: (i, 0)),
        ],
        out_shape=[
            jax.ShapeDtypeStruct((NPAD, HID), jnp.float32),
            jax.ShapeDtypeStruct((NPAD, 1), jnp.float32),
            jax.ShapeDtypeStruct((NPAD, 1), jnp.float32),
            jax.ShapeDtypeStruct((1, HID), jnp.float32),
            jax.ShapeDtypeStruct((1, 1), jnp.float32),
            jax.ShapeDtypeStruct((1, 1), jnp.float32),
            jax.ShapeDtypeStruct((1, 1), jnp.float32),
            jax.ShapeDtypeStruct((EPAD // 128, 128), jnp.float32),
        ],
    )(xp, w, att3, rel_emb, wrel, conf2d)


def _score_body(s1_hbm, s2_hbm, rel_hbm, shift_hbm, dst_hbm, src_hbm,
                t_hbm, clog_hbm, e_hbm, sp_hbm,
                s1v, s2v, relv, shv, dstv, srcv, tv, clogv, ev, zv, s_acc):
    cid = lax.axis_index("c")
    sid = lax.axis_index("s")
    wid = cid * NS + sid

    pltpu.sync_copy(s1_hbm, s1v)
    pltpu.sync_copy(s2_hbm, s2v)
    pltpu.sync_copy(rel_hbm, relv)
    pltpu.sync_copy(shift_hbm, shv)
    pltpu.sync_copy(dst_hbm.at[wid], dstv)
    pltpu.sync_copy(src_hbm.at[wid], srcv)
    pltpu.sync_copy(t_hbm.at[wid], tv)
    pltpu.sync_copy(clog_hbm.at[wid], clogv)

    def zfill(i, _):
        zv[pl.ds(i * L, L)] = jnp.zeros((L,), jnp.float32)
        return 0
    lax.fori_loop(0, SLICE // L, zfill, 0)
    pltpu.sync_copy(zv, s_acc.at[pl.ds(sid * SLICE, SLICE)])
    plsc.subcore_barrier()

    sh = shv[...]
    lane = jnp.arange(L, dtype=jnp.int32)

    def row(r, _):
        for c in range(CW // L):
            sl = pl.ds(c * L, L)
            d16 = dstv[r, sl]
            g = (plsc.load_gather(s1v, [d16])
                 + plsc.load_gather(s2v, [srcv[r, sl]])
                 + plsc.load_gather(relv, [tv[r, sl]]))
            g = jnp.where(g >= 0.0, g, 0.2 * g)
            g = g + clogv[r, sl] - sh
            e16 = jnp.exp(g)
            gidx = wid * EPT + r * CW + c * L + lane
            ev[r, sl] = jnp.where(gidx < E, e16, 0.0)
        pltpu.sync_copy(ev.at[r], s_acc.at[dstv.at[r]], add=True)
        return 0
    lax.fori_loop(0, CH, row, 0)

    plsc.subcore_barrier()
    pltpu.sync_copy(ev, e_hbm.at[wid])
    pltpu.sync_copy(s_acc.at[pl.ds(sid * SLICE, SLICE)],
                    sp_hbm.at[cid, pl.ds(sid * SLICE, SLICE)])


def _stage3(s1, s2, rel16, shift8, dstI, srcI, tI, clogI):
    mesh = plsc.VectorSubcoreMesh(core_axis_name="c", subcore_axis_name="s",
                                  num_cores=NC, num_subcores=NS)
    f = pl.kernel(
        _score_body,
        out_type=[
            jax.ShapeDtypeStruct((NW, CH, CW), jnp.float32),
            jax.ShapeDtypeStruct((NC, NPAD), jnp.float32),
        ],
        mesh=mesh,
        compiler_params=pltpu.CompilerParams(needs_layout_passes=False),
        scratch_types=[
            pltpu.VMEM((NPAD,), jnp.float32),
            pltpu.VMEM((NPAD,), jnp.float32),
            pltpu.VMEM((NUM_RELS,), jnp.float32),
            pltpu.VMEM((L,), jnp.float32),
            pltpu.VMEM((CH, CW), jnp.int32),
            pltpu.VMEM((CH, CW), jnp.int32),
            pltpu.VMEM((CH, CW), jnp.int32),
            pltpu.VMEM((CH, CW), jnp.float32),
            pltpu.VMEM((CH, CW), jnp.float32),
            pltpu.VMEM((SLICE,), jnp.float32),
            pltpu.VMEM_SHARED((NPAD,), jnp.float32),
        ],
    )
    return f(s1, s2, rel16, shift8, dstI, srcI, tI, clogI)


def _alpha_body(sp_hbm, dst_hbm, e_hbm, a_hbm, sp0v, sp1v, dstv, ev):
    cid = lax.axis_index("c")
    sid = lax.axis_index("s")
    wid = cid * NS + sid

    pltpu.sync_copy(sp_hbm.at[0], sp0v)
    pltpu.sync_copy(sp_hbm.at[1], sp1v)
    pltpu.sync_copy(dst_hbm.at[wid], dstv)
    pltpu.sync_copy(e_hbm.at[wid], ev)

    def comb(i, _):
        sl = pl.ds(i * L, L)
        sp0v[sl] = sp0v[sl] + sp1v[sl]
        return 0
    lax.fori_loop(0, NPAD // L, comb, 0)

    def arow(r, _):
        for c in range(CW // L):
            sl = pl.ds(c * L, L)
            s16 = plsc.load_gather(sp0v, [dstv[r, sl]])
            ev[r, sl] = ev[r, sl] / (s16 + 1e-16)
        return 0
    lax.fori_loop(0, CH, arow, 0)
    pltpu.sync_copy(ev, a_hbm.at[wid])


def _stage3b(sp, dstI, e):
    mesh = plsc.VectorSubcoreMesh(core_axis_name="c", subcore_axis_name="s",
                                  num_cores=NC, num_subcores=NS)
    f = pl.kernel(
        _alpha_body,
        out_type=jax.ShapeDtypeStruct((NW, CH, CW), jnp.float32),
        mesh=mesh,
        compiler_params=pltpu.CompilerParams(needs_layout_passes=False),
        scratch_types=[
            pltpu.VMEM((NPAD,), jnp.float32),
            pltpu.VMEM((NPAD,), jnp.float32),
            pltpu.VMEM((CH, CW), jnp.int32),
            pltpu.VMEM((CH, CW), jnp.float32),
        ],
    )
    return f(sp, dstI, e)


def _msg_body(h_hbm, a_hbm, dst_hbm, src_hbm, op_hbm,
              dpv, spv, apv, rows_a, rows_b, sem_a, sem_b, out_acc):
    cid = lax.axis_index("c")
    sid = lax.axis_index("s")
    wid = cid * NS + sid

    def zrow(k, _):
        for c in range(HID // L):
            rows_a[k, pl.ds(c * L, L)] = jnp.zeros((L,), jnp.float32)
        return 0
    lax.fori_loop(0, CW, zrow, 0)

    def zacc(j, _):
        pltpu.sync_copy(rows_a, out_acc.at[pl.ds(sid * SLICE + j * CW, CW)])
        return 0
    lax.fori_loop(0, SLICE // CW, zacc, 0)
    plsc.subcore_barrier()

    def run(chunk0, nch):
        for p in range(nch // PP):
            start = pl.multiple_of(chunk0 + p * PP, PP)
            pltpu.sync_copy(dst_hbm.at[pl.ds(start, PP)], dpv)
            pltpu.sync_copy(src_hbm.at[pl.ds(start, PP)], spv)
            pltpu.sync_copy(a_hbm.at[pl.ds(start, PP)], apv)
            pltpu.async_copy(h_hbm.at[spv.at[0]], rows_a, sem_a)
            pltpu.async_copy(h_hbm.at[spv.at[1]], rows_b, sem_b)

            def pair(g, _):
                for b in range(2):
                    rows = rows_a if b == 0 else rows_b
                    sem = sem_a if b == 0 else sem_b
                    r = g * 2 + b
                    pltpu.make_async_copy(h_hbm.at[spv.at[r]], rows, sem).wait()

                    def scale(k, _2):
                        a = plsc.load_gather(
                            apv, [jnp.full((L,), r, jnp.int32),
                                  jnp.full((L,), k, jnp.int32)])
                        for c in range(HID // L):
                            sl = pl.ds(c * L, L)
                            rows[k, sl] = rows[k, sl] * a
                        return 0
                    lax.fori_loop(0, CW, scale, 0)
                    pltpu.sync_copy(rows, out_acc.at[dpv.at[r]], add=True)

                    @pl.when(r + 2 < PP)
                    def _():
                        pltpu.async_copy(h_hbm.at[spv.at[r + 2]], rows, sem)
                return 0
            lax.fori_loop(0, PP // 2, pair, 0)

    @pl.when(cid == 0)
    def _():
        run(sid * CH_A, CH_A)

    @pl.when(cid == 1)
    def _():
        run(NS * CH_A + sid * CH_B, CH_B)

    plsc.subcore_barrier()
    pltpu.sync_copy(out_acc.at[pl.ds(sid * SLICE, SLICE)],
                    op_hbm.at[cid, pl.ds(sid * SLICE, SLICE)])


def _stage4(h, alpha, dstI, srcI):
    mesh = plsc.VectorSubcoreMesh(core_axis_name="c", subcore_axis_name="s",
                                  num_cores=NC, num_subcores=NS)
    f = pl.kernel(
        _msg_body,
        out_type=jax.ShapeDtypeStruct((NC, NPAD, HID), jnp.float32),
        mesh=mesh,
        compiler_params=pltpu.CompilerParams(needs_layout_passes=False),
        scratch_types=[
            pltpu.VMEM((PP, CW), jnp.int32),
            pltpu.VMEM((PP, CW), jnp.int32),
            pltpu.VMEM((PP, CW), jnp.float32),
            pltpu.VMEM((CW, HID), jnp.float32),
            pltpu.VMEM((CW, HID), jnp.float32),
            pltpu.SemaphoreType.DMA,
            pltpu.SemaphoreType.DMA,
            pltpu.VMEM_SHARED((NPAD, HID), jnp.float32),
        ],
    )
    return f(h, alpha.reshape(EPAD // CW, CW), dstI.reshape(EPAD // CW, CW),
             srcI.reshape(EPAD // CW, CW))


def _fin_body(p0_ref, p1_ref, b_ref, o_ref):
    o_ref[...] = p0_ref[...] + p1_ref[...] + b_ref[...]


def _stage5(p0, p1, bias2d):
    return pl.pallas_call(
        _fin_body,
        grid=(NS,),
        in_specs=[
            pl.BlockSpec((SLICE, HID), lambda i: (i, 0)),
            pl.BlockSpec((SLICE, HID), lambda i: (i, 0)),
            pl.BlockSpec((1, HID), lambda i: (0, 0)),
        ],
        out_specs=pl.BlockSpec((SLICE, HID), lambda i: (i, 0)),
        out_shape=jax.ShapeDtypeStruct((NPAD, HID), jnp.float32),
    )(p0, p1, bias2d)


@jax.jit
def kernel(x, edge_index, edge_type_in, edge_attr, W_msg, rel_emb, W_rel,
           att_vec, bias):
    src = edge_index[0].astype(jnp.int32)
    dst = edge_index[1].astype(jnp.int32)
    t = jnp.clip(edge_type_in, 0, NUM_RELS - 1).astype(jnp.int32)
    conf = edge_attr[:, 0].astype(jnp.float32)

    pad = EPAD - E
    srcI = jnp.pad(src, (0, pad)).reshape(NW, CH, CW)
    dstI = jnp.pad(dst, (0, pad)).reshape(NW, CH, CW)
    tI = jnp.pad(t, (0, pad)).reshape(NW, CH, CW)
    confP = jnp.pad(conf, (0, pad), constant_values=1.0)

    xp = jnp.pad(x, ((0, NPAD - N), (0, 0)))
    att3 = att_vec.reshape(3, HID)

    h, s1, s2, rs, m1, m2, m3, clog2d = _stage1(
        xp, W_msg, att3, rel_emb, W_rel, confP.reshape(EPAD // 128, 128))
    shift = jnp.maximum(m1[0, 0] + m2[0, 0] + m3[0, 0], 0.0)
    shift16 = jnp.broadcast_to(shift, (L,))
    clog = clog2d.reshape(NW, CH, CW)

    e, sp = _stage3(s1.reshape(NPAD), s2.reshape(NPAD), rs[0, :NUM_RELS],
                    shift16, dstI, srcI, tI, clog)
    alpha = _stage3b(sp, dstI, e)
    op = _stage4(h, alpha, dstI, srcI)
    out = _stage5(op[0], op[1], bias.reshape(1, HID))
    return out[:N]


# speedup vs baseline: 1.2003x; 1.0218x over previous
import functools

import jax
import jax.numpy as jnp
from jax import lax
from jax.experimental import pallas as pl
from jax.experimental.pallas import tpu as pltpu
from jax.experimental.pallas import tpu_sc as plsc

HID = 128
NUM_RELS = 16
N = 10000
NPAD = 10240
E = 320000
NC, NS, L = 2, 16, 16
NW = NC * NS
CW = 128
CH = 80
PP = 8
CH_A = 128
CH_B = 32
EPT = CH * CW
EPAD = NW * EPT
SLICE = NPAD // NS
NB = 512


def _node_body(x_ref, w_ref, att_ref, rel_ref, wrel_ref, c_ref,
               h_ref, s1_ref, s2_ref, rs_ref, m1_ref, m2_ref, m3_ref,
               cl_ref):
    i = pl.program_id(0)
    cl_ref[...] = 0.1 * jnp.log(jnp.maximum(c_ref[...], 1e-6))
    h = lax.dot_general(x_ref[...], w_ref[...], (((1,), (1,)), ((), ())),
                        preferred_element_type=jnp.float32)
    h_ref[...] = h
    a_dst = att_ref[0:1, :]
    a_src = att_ref[1:2, :]
    s1 = lax.dot_general(h, a_dst, (((1,), (1,)), ((), ())),
                         preferred_element_type=jnp.float32)
    s2 = lax.dot_general(h, a_src, (((1,), (1,)), ((), ())),
                         preferred_element_type=jnp.float32)
    s1_ref[...] = s1
    s2_ref[...] = s2
    neg = jnp.full((1, 1), -jnp.inf, jnp.float32)
    b1 = jnp.max(s1, keepdims=True)
    b2 = jnp.max(s2, keepdims=True)
    m1_ref[...] = jnp.maximum(jnp.where(i == 0, neg, m1_ref[...]), b1)
    m2_ref[...] = jnp.maximum(jnp.where(i == 0, neg, m2_ref[...]), b2)

    @pl.when(i == 0)
    def _():
        a_rel = att_ref[2:3, :]
        c = lax.dot_general(a_rel, wrel_ref[...], (((1,), (0,)), ((), ())),
                            preferred_element_type=jnp.float32)
        rs = lax.dot_general(rel_ref[...], c, (((1,), (1,)), ((), ())),
                             preferred_element_type=jnp.float32)
        rs_row = rs.reshape(1, 16)
        rs_ref[...] = jnp.concatenate(
            [rs_row, jnp.zeros((1, 112), jnp.float32)], axis=1)
        m3_ref[...] = jnp.max(rs, keepdims=True)


def _stage1(xp, w, att3, rel_emb, wrel, conf2d):
    grid = NPAD // NB
    return pl.pallas_call(
        _node_body,
        grid=(grid,),
        in_specs=[
            pl.BlockSpec((NB, HID), lambda i: (i, 0)),
            pl.BlockSpec((HID, HID), lambda i: (0, 0)),
            pl.BlockSpec((3, HID), lambda i: (0, 0)),
            pl.BlockSpec((NUM_RELS, NUM_RELS), lambda i: (0, 0)),
            pl.BlockSpec((HID, NUM_RELS), lambda i: (0, 0)),
            pl.BlockSpec((CW, 128), lambda i: (i, 0)),
        ],
        out_specs=[
            pl.BlockSpec((NB, HID), lambda i: (i, 0)),
            pl.BlockSpec((NB, 1), lambda i: (i, 0)),
            pl.BlockSpec((NB, 1), lambda i: (i, 0)),
            pl.BlockSpec((1, HID), lambda i: (0, 0)),
            pl.BlockSpec((1, 1), lambda i: (0, 0)),
            pl.BlockSpec((1, 1), lambda i: (0, 0)),
            pl.BlockSpec((1, 1), lambda i: (0, 0)),
            pl.BlockSpec((CW, 128), lambda i: (i, 0)),
        ],
        out_shape=[
            jax.ShapeDtypeStruct((NPAD, HID), jnp.float32),
            jax.ShapeDtypeStruct((NPAD, 1), jnp.float32),
            jax.ShapeDtypeStruct((NPAD, 1), jnp.float32),
            jax.ShapeDtypeStruct((1, HID), jnp.float32),
            jax.ShapeDtypeStruct((1, 1), jnp.float32),
            jax.ShapeDtypeStruct((1, 1), jnp.float32),
            jax.ShapeDtypeStruct((1, 1), jnp.float32),
            jax.ShapeDtypeStruct((EPAD // 128, 128), jnp.float32),
        ],
    )(xp, w, att3, rel_emb, wrel, conf2d)


def _score_body(s1_hbm, s2_hbm, rel_hbm, shift_hbm, dst_hbm, src_hbm,
                t_hbm, clog_hbm, e_hbm, sp_hbm,
                s1v, s2v, relv, shv, dstv, srcv, tv, clogv, ev, zv, s_acc):
    cid = lax.axis_index("c")
    sid = lax.axis_index("s")
    wid = cid * NS + sid

    pltpu.sync_copy(s1_hbm, s1v)
    pltpu.sync_copy(s2_hbm, s2v)
    pltpu.sync_copy(rel_hbm, relv)
    pltpu.sync_copy(shift_hbm, shv)
    pltpu.sync_copy(dst_hbm.at[wid], dstv)
    pltpu.sync_copy(src_hbm.at[wid], srcv)
    pltpu.sync_copy(t_hbm.at[wid], tv)
    pltpu.sync_copy(clog_hbm.at[wid], clogv)

    def zfill(i, _):
        zv[pl.ds(i * L, L)] = jnp.zeros((L,), jnp.float32)
        return 0
    lax.fori_loop(0, SLICE // L, zfill, 0)
    pltpu.sync_copy(zv, s_acc.at[pl.ds(sid * SLICE, SLICE)])
    plsc.subcore_barrier()

    sh = shv[...]
    lane = jnp.arange(L, dtype=jnp.int32)

    def row(r, _):
        for c in range(CW // L):
            sl = pl.ds(c * L, L)
            d16 = dstv[r, sl]
            g = (plsc.load_gather(s1v, [d16])
                 + plsc.load_gather(s2v, [srcv[r, sl]])
                 + plsc.load_gather(relv, [tv[r, sl]]))
            g = jnp.where(g >= 0.0, g, 0.2 * g)
            g = g + clogv[r, sl] - sh
            e16 = jnp.exp(g)
            gidx = wid * EPT + r * CW + c * L + lane
            ev[r, sl] = jnp.where(gidx < E, e16, 0.0)
        pltpu.sync_copy(ev.at[r], s_acc.at[dstv.at[r]], add=True)
        return 0
    lax.fori_loop(0, CH, row, 0)

    plsc.subcore_barrier()
    pltpu.sync_copy(ev, e_hbm.at[wid])
    pltpu.sync_copy(s_acc.at[pl.ds(sid * SLICE, SLICE)],
                    sp_hbm.at[cid, pl.ds(sid * SLICE, SLICE)])


def _stage3(s1, s2, rel16, shift8, dstI, srcI, tI, clogI):
    mesh = plsc.VectorSubcoreMesh(core_axis_name="c", subcore_axis_name="s",
                                  num_cores=NC, num_subcores=NS)
    f = pl.kernel(
        _score_body,
        out_type=[
            jax.ShapeDtypeStruct((NW, CH, CW), jnp.float32),
            jax.ShapeDtypeStruct((NC, NPAD), jnp.float32),
        ],
        mesh=mesh,
        compiler_params=pltpu.CompilerParams(needs_layout_passes=False),
        scratch_types=[
            pltpu.VMEM((NPAD,), jnp.float32),
            pltpu.VMEM((NPAD,), jnp.float32),
            pltpu.VMEM((NUM_RELS,), jnp.float32),
            pltpu.VMEM((L,), jnp.float32),
            pltpu.VMEM((CH, CW), jnp.int32),
            pltpu.VMEM((CH, CW), jnp.int32),
            pltpu.VMEM((CH, CW), jnp.int32),
            pltpu.VMEM((CH, CW), jnp.float32),
            pltpu.VMEM((CH, CW), jnp.float32),
            pltpu.VMEM((SLICE,), jnp.float32),
            pltpu.VMEM_SHARED((NPAD,), jnp.float32),
        ],
    )
    return f(s1, s2, rel16, shift8, dstI, srcI, tI, clogI)


def _alpha_body(sp_hbm, dst_hbm, e_hbm, a_hbm, sp0v, sp1v, dstv, ev):
    cid = lax.axis_index("c")
    sid = lax.axis_index("s")
    wid = cid * NS + sid

    pltpu.sync_copy(sp_hbm.at[0], sp0v)
    pltpu.sync_copy(sp_hbm.at[1], sp1v)
    pltpu.sync_copy(dst_hbm.at[wid], dstv)
    pltpu.sync_copy(e_hbm.at[wid], ev)

    def comb(i, _):
        sl = pl.ds(i * L, L)
        sp0v[sl] = sp0v[sl] + sp1v[sl]
        return 0
    lax.fori_loop(0, NPAD // L, comb, 0)

    def arow(r, _):
        for c in range(CW // L):
            sl = pl.ds(c * L, L)
            s16 = plsc.load_gather(sp0v, [dstv[r, sl]])
            ev[r, sl] = ev[r, sl] / (s16 + 1e-16)
        return 0
    lax.fori_loop(0, CH, arow, 0)
    pltpu.sync_copy(ev, a_hbm.at[wid])


def _stage3b(sp, dstI, e):
    mesh = plsc.VectorSubcoreMesh(core_axis_name="c", subcore_axis_name="s",
                                  num_cores=NC, num_subcores=NS)
    f = pl.kernel(
        _alpha_body,
        out_type=jax.ShapeDtypeStruct((NW, CH, CW), jnp.float32),
        mesh=mesh,
        compiler_params=pltpu.CompilerParams(needs_layout_passes=False),
        scratch_types=[
            pltpu.VMEM((NPAD,), jnp.float32),
            pltpu.VMEM((NPAD,), jnp.float32),
            pltpu.VMEM((CH, CW), jnp.int32),
            pltpu.VMEM((CH, CW), jnp.float32),
        ],
    )
    return f(sp, dstI, e)


def _msg_body(h_hbm, a_hbm, dst_hbm, src_hbm, op_hbm,
              dpv, spv, apv, rows_a, rows_b, sem_a, sem_b, out_acc):
    cid = lax.axis_index("c")
    sid = lax.axis_index("s")
    wid = cid * NS + sid

    def zrow(k, _):
        for c in range(HID // L):
            rows_a[k, pl.ds(c * L, L)] = jnp.zeros((L,), jnp.float32)
        return 0
    lax.fori_loop(0, CW, zrow, 0)

    def zacc(j, _):
        pltpu.sync_copy(rows_a, out_acc.at[pl.ds(sid * SLICE + j * CW, CW)])
        return 0
    lax.fori_loop(0, SLICE // CW, zacc, 0)
    plsc.subcore_barrier()

    def run(chunk0, nch):
        for p in range(nch // PP):
            start = pl.multiple_of(chunk0 + p * PP, PP)
            pltpu.sync_copy(dst_hbm.at[pl.ds(start, PP)], dpv)
            pltpu.sync_copy(src_hbm.at[pl.ds(start, PP)], spv)
            pltpu.sync_copy(a_hbm.at[pl.ds(start, PP)], apv)
            pltpu.async_copy(h_hbm.at[spv.at[0]], rows_a, sem_a)
            pltpu.async_copy(h_hbm.at[spv.at[1]], rows_b, sem_b)

            def pair(g, _):
                for b in range(2):
                    rows = rows_a if b == 0 else rows_b
                    sem = sem_a if b == 0 else sem_b
                    r = g * 2 + b
                    pltpu.make_async_copy(h_hbm.at[spv.at[r]], rows, sem).wait()

                    def scale(k, _2):
                        a = plsc.load_gather(
                            apv, [jnp.full((L,), r, jnp.int32),
                                  jnp.full((L,), k, jnp.int32)])
                        for c in range(HID // L):
                            sl = pl.ds(c * L, L)
                            rows[k, sl] = rows[k, sl] * a
                        return 0
                    lax.fori_loop(0, CW, scale, 0)
                    pltpu.sync_copy(rows, out_acc.at[dpv.at[r]], add=True)

                    @pl.when(r + 2 < PP)
                    def _():
                        pltpu.async_copy(h_hbm.at[spv.at[r + 2]], rows, sem)
                return 0
            lax.fori_loop(0, PP // 2, pair, 0)

    @pl.when(cid == 0)
    def _():
        run(sid * CH_A, CH_A)

    @pl.when(cid == 1)
    def _():
        run(NS * CH_A + sid * CH_B, CH_B)

    plsc.subcore_barrier()
    pltpu.sync_copy(out_acc.at[pl.ds(sid * SLICE, SLICE)],
                    op_hbm.at[cid, pl.ds(sid * SLICE, SLICE)])


def _stage4(h, alpha, dstI, srcI):
    mesh = plsc.VectorSubcoreMesh(core_axis_name="c", subcore_axis_name="s",
                                  num_cores=NC, num_subcores=NS)
    f = pl.kernel(
        _msg_body,
        out_type=jax.ShapeDtypeStruct((NC, NPAD, HID), jnp.float32),
        mesh=mesh,
        compiler_params=pltpu.CompilerParams(needs_layout_passes=False),
        scratch_types=[
            pltpu.VMEM((PP, CW), jnp.int32),
            pltpu.VMEM((PP, CW), jnp.int32),
            pltpu.VMEM((PP, CW), jnp.float32),
            pltpu.VMEM((CW, HID), jnp.float32),
            pltpu.VMEM((CW, HID), jnp.float32),
            pltpu.SemaphoreType.DMA,
            pltpu.SemaphoreType.DMA,
            pltpu.VMEM_SHARED((NPAD, HID), jnp.float32),
        ],
    )
    return f(h, alpha.reshape(EPAD // CW, CW), dstI.reshape(EPAD // CW, CW),
             srcI.reshape(EPAD // CW, CW))


def _fin_body(p0_ref, p1_ref, b_ref, o_ref):
    o_ref[...] = p0_ref[...] + p1_ref[...] + b_ref[...]


def _stage5(p0, p1, bias2d):
    return pl.pallas_call(
        _fin_body,
        grid=(NS,),
        in_specs=[
            pl.BlockSpec((SLICE, HID), lambda i: (i, 0)),
            pl.BlockSpec((SLICE, HID), lambda i: (i, 0)),
            pl.BlockSpec((1, HID), lambda i: (0, 0)),
        ],
        out_specs=pl.BlockSpec((SLICE, HID), lambda i: (i, 0)),
        out_shape=jax.ShapeDtypeStruct((NPAD, HID), jnp.float32),
    )(p0, p1, bias2d)


@jax.jit
def kernel(x, edge_index, edge_type_in, edge_attr, W_msg, rel_emb, W_rel,
           att_vec, bias):
    src = edge_index[0].astype(jnp.int32)
    dst = edge_index[1].astype(jnp.int32)
    t = jnp.clip(edge_type_in, 0, NUM_RELS - 1).astype(jnp.int32)
    conf = edge_attr[:, 0].astype(jnp.float32)

    pad = EPAD - E
    srcI = jnp.pad(src, (0, pad)).reshape(NW, CH, CW)
    dstI = jnp.pad(dst, (0, pad)).reshape(NW, CH, CW)
    tI = jnp.pad(t, (0, pad)).reshape(NW, CH, CW)
    confP = jnp.pad(conf, (0, pad), constant_values=1.0)

    xp = jnp.pad(x, ((0, NPAD - N), (0, 0)))
    att3 = att_vec.reshape(3, HID)

    h, s1, s2, rs, m1, m2, m3, clog2d = _stage1(
        xp, W_msg, att3, rel_emb, W_rel, confP.reshape(EPAD // 128, 128))
    shift = jnp.maximum(m1[0, 0] + m2[0, 0] + m3[0, 0], 0.0)
    shift16 = jnp.broadcast_to(shift, (L,))
    clog = clog2d.reshape(NW, CH, CW)

    e, sp = _stage3(s1.reshape(NPAD), s2.reshape(NPAD), rs[0, :NUM_RELS],
                    shift16, dstI, srcI, tI, clog)
    alpha = _stage3b(sp, dstI, e)
    op = _stage4(h, alpha, dstI, srcI)
    out = _stage5(op[0], op[1], bias.reshape(1, HID))
    return out[:N]


# speedup vs baseline: 1.2008x; 1.0004x over previous
import functools

import jax
import jax.numpy as jnp
from jax import lax
from jax.experimental import pallas as pl
from jax.experimental.pallas import tpu as pltpu
from jax.experimental.pallas import tpu_sc as plsc

HID = 128
NUM_RELS = 16
N = 10000
NPAD = 10240
E = 320000
NC, NS, L = 2, 16, 16
NW = NC * NS
CW = 128
CH = 80
PP = 8
CH_A = 136
CH_B = 24
EPT = CH * CW
EPAD = NW * EPT
SLICE = NPAD // NS
NB = 512


def _node_body(x_ref, w_ref, att_ref, rel_ref, wrel_ref, c_ref,
               h_ref, s1_ref, s2_ref, rs_ref, m1_ref, m2_ref, m3_ref,
               cl_ref):
    i = pl.program_id(0)
    cl_ref[...] = 0.1 * jnp.log(jnp.maximum(c_ref[...], 1e-6))
    h = lax.dot_general(x_ref[...], w_ref[...], (((1,), (1,)), ((), ())),
                        preferred_element_type=jnp.float32)
    h_ref[...] = h
    a_dst = att_ref[0:1, :]
    a_src = att_ref[1:2, :]
    s1 = lax.dot_general(h, a_dst, (((1,), (1,)), ((), ())),
                         preferred_element_type=jnp.float32)
    s2 = lax.dot_general(h, a_src, (((1,), (1,)), ((), ())),
                         preferred_element_type=jnp.float32)
    s1_ref[...] = s1
    s2_ref[...] = s2
    neg = jnp.full((1, 1), -jnp.inf, jnp.float32)
    b1 = jnp.max(s1, keepdims=True)
    b2 = jnp.max(s2, keepdims=True)
    m1_ref[...] = jnp.maximum(jnp.where(i == 0, neg, m1_ref[...]), b1)
    m2_ref[...] = jnp.maximum(jnp.where(i == 0, neg, m2_ref[...]), b2)

    @pl.when(i == 0)
    def _():
        a_rel = att_ref[2:3, :]
        c = lax.dot_general(a_rel, wrel_ref[...], (((1,), (0,)), ((), ())),
                            preferred_element_type=jnp.float32)
        rs = lax.dot_general(rel_ref[...], c, (((1,), (1,)), ((), ())),
                             preferred_element_type=jnp.float32)
        rs_row = rs.reshape(1, 16)
        rs_ref[...] = jnp.concatenate(
            [rs_row, jnp.zeros((1, 112), jnp.float32)], axis=1)
        m3_ref[...] = jnp.max(rs, keepdims=True)


def _stage1(xp, w, att3, rel_emb, wrel, conf2d):
    grid = NPAD // NB
    return pl.pallas_call(
        _node_body,
        grid=(grid,),
        in_specs=[
            pl.BlockSpec((NB, HID), lambda i: (i, 0)),
            pl.BlockSpec((HID, HID), lambda i: (0, 0)),
            pl.BlockSpec((3, HID), lambda i: (0, 0)),
            pl.BlockSpec((NUM_RELS, NUM_RELS), lambda i: (0, 0)),
            pl.BlockSpec((HID, NUM_RELS), lambda i: (0, 0)),
            pl.BlockSpec((CW, 128), lambda i: (i, 0)),
        ],
        out_specs=[
            pl.BlockSpec((NB, HID), lambda i: (i, 0)),
            pl.BlockSpec((NB, 1), lambda i: (i, 0)),
            pl.BlockSpec((NB, 1), lambda i: (i, 0)),
            pl.BlockSpec((1, HID), lambda i: (0, 0)),
            pl.BlockSpec((1, 1), lambda i: (0, 0)),
            pl.BlockSpec((1, 1), lambda i: (0, 0)),
            pl.BlockSpec((1, 1), lambda i: (0, 0)),
            pl.BlockSpec((CW, 128), lambda i: (i, 0)),
        ],
        out_shape=[
            jax.ShapeDtypeStruct((NPAD, HID), jnp.float32),
            jax.ShapeDtypeStruct((NPAD, 1), jnp.float32),
            jax.ShapeDtypeStruct((NPAD, 1), jnp.float32),
            jax.ShapeDtypeStruct((1, HID), jnp.float32),
            jax.ShapeDtypeStruct((1, 1), jnp.float32),
            jax.ShapeDtypeStruct((1, 1), jnp.float32),
            jax.ShapeDtypeStruct((1, 1), jnp.float32),
            jax.ShapeDtypeStruct((EPAD // 128, 128), jnp.float32),
        ],
    )(xp, w, att3, rel_emb, wrel, conf2d)


def _score_body(s1_hbm, s2_hbm, rel_hbm, shift_hbm, dst_hbm, src_hbm,
                t_hbm, clog_hbm, e_hbm, sp_hbm,
                s1v, s2v, relv, shv, dstv, srcv, tv, clogv, ev, zv, s_acc):
    cid = lax.axis_index("c")
    sid = lax.axis_index("s")
    wid = cid * NS + sid

    pltpu.sync_copy(s1_hbm, s1v)
    pltpu.sync_copy(s2_hbm, s2v)
    pltpu.sync_copy(rel_hbm, relv)
    pltpu.sync_copy(shift_hbm, shv)
    pltpu.sync_copy(dst_hbm.at[wid], dstv)
    pltpu.sync_copy(src_hbm.at[wid], srcv)
    pltpu.sync_copy(t_hbm.at[wid], tv)
    pltpu.sync_copy(clog_hbm.at[wid], clogv)

    def zfill(i, _):
        zv[pl.ds(i * L, L)] = jnp.zeros((L,), jnp.float32)
        return 0
    lax.fori_loop(0, SLICE // L, zfill, 0)
    pltpu.sync_copy(zv, s_acc.at[pl.ds(sid * SLICE, SLICE)])
    plsc.subcore_barrier()

    sh = shv[...]
    lane = jnp.arange(L, dtype=jnp.int32)

    def row(r, _):
        for c in range(CW // L):
            sl = pl.ds(c * L, L)
            d16 = dstv[r, sl]
            g = (plsc.load_gather(s1v, [d16])
                 + plsc.load_gather(s2v, [srcv[r, sl]])
                 + plsc.load_gather(relv, [tv[r, sl]]))
            g = jnp.where(g >= 0.0, g, 0.2 * g)
            g = g + clogv[r, sl] - sh
            e16 = jnp.exp(g)
            gidx = wid * EPT + r * CW + c * L + lane
            ev[r, sl] = jnp.where(gidx < E, e16, 0.0)
        pltpu.sync_copy(ev.at[r], s_acc.at[dstv.at[r]], add=True)
        return 0
    lax.fori_loop(0, CH, row, 0)

    plsc.subcore_barrier()
    pltpu.sync_copy(ev, e_hbm.at[wid])
    pltpu.sync_copy(s_acc.at[pl.ds(sid * SLICE, SLICE)],
                    sp_hbm.at[cid, pl.ds(sid * SLICE, SLICE)])


def _stage3(s1, s2, rel16, shift8, dstI, srcI, tI, clogI):
    mesh = plsc.VectorSubcoreMesh(core_axis_name="c", subcore_axis_name="s",
                                  num_cores=NC, num_subcores=NS)
    f = pl.kernel(
        _score_body,
        out_type=[
            jax.ShapeDtypeStruct((NW, CH, CW), jnp.float32),
            jax.ShapeDtypeStruct((NC, NPAD), jnp.float32),
        ],
        mesh=mesh,
        compiler_params=pltpu.CompilerParams(needs_layout_passes=False),
        scratch_types=[
            pltpu.VMEM((NPAD,), jnp.float32),
            pltpu.VMEM((NPAD,), jnp.float32),
            pltpu.VMEM((NUM_RELS,), jnp.float32),
            pltpu.VMEM((L,), jnp.float32),
            pltpu.VMEM((CH, CW), jnp.int32),
            pltpu.VMEM((CH, CW), jnp.int32),
            pltpu.VMEM((CH, CW), jnp.int32),
            pltpu.VMEM((CH, CW), jnp.float32),
            pltpu.VMEM((CH, CW), jnp.float32),
            pltpu.VMEM((SLICE,), jnp.float32),
            pltpu.VMEM_SHARED((NPAD,), jnp.float32),
        ],
    )
    return f(s1, s2, rel16, shift8, dstI, srcI, tI, clogI)


def _alpha_body(sp_hbm, dst_hbm, e_hbm, a_hbm, sp0v, sp1v, dstv, ev):
    cid = lax.axis_index("c")
    sid = lax.axis_index("s")
    wid = cid * NS + sid

    pltpu.sync_copy(sp_hbm.at[0], sp0v)
    pltpu.sync_copy(sp_hbm.at[1], sp1v)
    pltpu.sync_copy(dst_hbm.at[wid], dstv)
    pltpu.sync_copy(e_hbm.at[wid], ev)

    def comb(i, _):
        sl = pl.ds(i * L, L)
        sp0v[sl] = sp0v[sl] + sp1v[sl]
        return 0
    lax.fori_loop(0, NPAD // L, comb, 0)

    def arow(r, _):
        for c in range(CW // L):
            sl = pl.ds(c * L, L)
            s16 = plsc.load_gather(sp0v, [dstv[r, sl]])
            ev[r, sl] = ev[r, sl] / (s16 + 1e-16)
        return 0
    lax.fori_loop(0, CH, arow, 0)
    pltpu.sync_copy(ev, a_hbm.at[wid])


def _stage3b(sp, dstI, e):
    mesh = plsc.VectorSubcoreMesh(core_axis_name="c", subcore_axis_name="s",
                                  num_cores=NC, num_subcores=NS)
    f = pl.kernel(
        _alpha_body,
        out_type=jax.ShapeDtypeStruct((NW, CH, CW), jnp.float32),
        mesh=mesh,
        compiler_params=pltpu.CompilerParams(needs_layout_passes=False),
        scratch_types=[
            pltpu.VMEM((NPAD,), jnp.float32),
            pltpu.VMEM((NPAD,), jnp.float32),
            pltpu.VMEM((CH, CW), jnp.int32),
            pltpu.VMEM((CH, CW), jnp.float32),
        ],
    )
    return f(sp, dstI, e)


def _msg_body(h_hbm, a_hbm, dst_hbm, src_hbm, op_hbm,
              dpv, spv, apv, rows_a, rows_b, sem_a, sem_b, out_acc):
    cid = lax.axis_index("c")
    sid = lax.axis_index("s")
    wid = cid * NS + sid

    def zrow(k, _):
        for c in range(HID // L):
            rows_a[k, pl.ds(c * L, L)] = jnp.zeros((L,), jnp.float32)
        return 0
    lax.fori_loop(0, CW, zrow, 0)

    def zacc(j, _):
        pltpu.sync_copy(rows_a, out_acc.at[pl.ds(sid * SLICE + j * CW, CW)])
        return 0
    lax.fori_loop(0, SLICE // CW, zacc, 0)
    plsc.subcore_barrier()

    def run(chunk0, nch):
        for p in range(nch // PP):
            start = pl.multiple_of(chunk0 + p * PP, PP)
            pltpu.sync_copy(dst_hbm.at[pl.ds(start, PP)], dpv)
            pltpu.sync_copy(src_hbm.at[pl.ds(start, PP)], spv)
            pltpu.sync_copy(a_hbm.at[pl.ds(start, PP)], apv)
            pltpu.async_copy(h_hbm.at[spv.at[0]], rows_a, sem_a)
            pltpu.async_copy(h_hbm.at[spv.at[1]], rows_b, sem_b)

            def pair(g, _):
                for b in range(2):
                    rows = rows_a if b == 0 else rows_b
                    sem = sem_a if b == 0 else sem_b
                    r = g * 2 + b
                    pltpu.make_async_copy(h_hbm.at[spv.at[r]], rows, sem).wait()

                    def scale(k, _2):
                        a = plsc.load_gather(
                            apv, [jnp.full((L,), r, jnp.int32),
                                  jnp.full((L,), k, jnp.int32)])
                        for c in range(HID // L):
                            sl = pl.ds(c * L, L)
                            rows[k, sl] = rows[k, sl] * a
                        return 0
                    lax.fori_loop(0, CW, scale, 0)
                    pltpu.sync_copy(rows, out_acc.at[dpv.at[r]], add=True)

                    @pl.when(r + 2 < PP)
                    def _():
                        pltpu.async_copy(h_hbm.at[spv.at[r + 2]], rows, sem)
                return 0
            lax.fori_loop(0, PP // 2, pair, 0)

    @pl.when(cid == 0)
    def _():
        run(sid * CH_A, CH_A)

    @pl.when(cid == 1)
    def _():
        run(NS * CH_A + sid * CH_B, CH_B)

    plsc.subcore_barrier()
    pltpu.sync_copy(out_acc.at[pl.ds(sid * SLICE, SLICE)],
                    op_hbm.at[cid, pl.ds(sid * SLICE, SLICE)])


def _stage4(h, alpha, dstI, srcI):
    mesh = plsc.VectorSubcoreMesh(core_axis_name="c", subcore_axis_name="s",
                                  num_cores=NC, num_subcores=NS)
    f = pl.kernel(
        _msg_body,
        out_type=jax.ShapeDtypeStruct((NC, NPAD, HID), jnp.float32),
        mesh=mesh,
        compiler_params=pltpu.CompilerParams(needs_layout_passes=False),
        scratch_types=[
            pltpu.VMEM((PP, CW), jnp.int32),
            pltpu.VMEM((PP, CW), jnp.int32),
            pltpu.VMEM((PP, CW), jnp.float32),
            pltpu.VMEM((CW, HID), jnp.float32),
            pltpu.VMEM((CW, HID), jnp.float32),
            pltpu.SemaphoreType.DMA,
            pltpu.SemaphoreType.DMA,
            pltpu.VMEM_SHARED((NPAD, HID), jnp.float32),
        ],
    )
    return f(h, alpha.reshape(EPAD // CW, CW), dstI.reshape(EPAD // CW, CW),
             srcI.reshape(EPAD // CW, CW))


def _fin_body(p0_ref, p1_ref, b_ref, o_ref):
    o_ref[...] = p0_ref[...] + p1_ref[...] + b_ref[...]


def _stage5(p0, p1, bias2d):
    return pl.pallas_call(
        _fin_body,
        grid=(NS,),
        in_specs=[
            pl.BlockSpec((SLICE, HID), lambda i: (i, 0)),
            pl.BlockSpec((SLICE, HID), lambda i: (i, 0)),
            pl.BlockSpec((1, HID), lambda i: (0, 0)),
        ],
        out_specs=pl.BlockSpec((SLICE, HID), lambda i: (i, 0)),
        out_shape=jax.ShapeDtypeStruct((NPAD, HID), jnp.float32),
    )(p0, p1, bias2d)


@jax.jit
def kernel(x, edge_index, edge_type_in, edge_attr, W_msg, rel_emb, W_rel,
           att_vec, bias):
    src = edge_index[0].astype(jnp.int32)
    dst = edge_index[1].astype(jnp.int32)
    t = jnp.clip(edge_type_in, 0, NUM_RELS - 1).astype(jnp.int32)
    conf = edge_attr[:, 0].astype(jnp.float32)

    pad = EPAD - E
    srcI = jnp.pad(src, (0, pad)).reshape(NW, CH, CW)
    dstI = jnp.pad(dst, (0, pad)).reshape(NW, CH, CW)
    tI = jnp.pad(t, (0, pad)).reshape(NW, CH, CW)
    confP = jnp.pad(conf, (0, pad), constant_values=1.0)

    xp = jnp.pad(x, ((0, NPAD - N), (0, 0)))
    att3 = att_vec.reshape(3, HID)

    h, s1, s2, rs, m1, m2, m3, clog2d = _stage1(
        xp, W_msg, att3, rel_emb, W_rel, confP.reshape(EPAD // 128, 128))
    shift = jnp.maximum(m1[0, 0] + m2[0, 0] + m3[0, 0], 0.0)
    shift16 = jnp.broadcast_to(shift, (L,))
    clog = clog2d.reshape(NW, CH, CW)

    e, sp = _stage3(s1.reshape(NPAD), s2.reshape(NPAD), rs[0, :NUM_RELS],
                    shift16, dstI, srcI, tI, clog)
    alpha = _stage3b(sp, dstI, e)
    op = _stage4(h, alpha, dstI, srcI)
    out = _stage5(op[0], op[1], bias.reshape(1, HID))
    return out[:N]
